# baseline (device time: 86156 ns/iter reference)
import jax
import jax.numpy as jnp
from jax import lax
from jax.experimental import pallas as pl
from jax.experimental.pallas import tpu as pltpu

N_DEV = 8
M = 768
CHUNK = M // N_DEV
N_HOPS = 2 * (N_DEV - 1)


def kernel(x, W1, W2):
    m, k = x.shape
    _, h_per = W1.shape
    _, n = W2.shape

    def body(x_ref, w1_ref, w2_ref, out_ref, acc_ref, comm_ref,
             send_sems, recv_sems):
        my = lax.axis_index("i")
        left = (my + N_DEV - 1) % N_DEV
        right = (my + 1) % N_DEV

        barrier_sem = pltpu.get_barrier_semaphore()
        for nbr in (left, right):
            pl.semaphore_signal(
                barrier_sem, inc=1,
                device_id=(nbr,), device_id_type=pl.DeviceIdType.MESH,
            )
        pl.semaphore_wait(barrier_sem, 2)

        h = jnp.maximum(
            jnp.dot(x_ref[:, :], w1_ref[:, :],
                    preferred_element_type=jnp.float32),
            0.0,
        )
        acc_ref[:, :] = jnp.dot(h, w2_ref[:, :],
                                preferred_element_type=jnp.float32)

        stage = N_HOPS
        c0 = (my + N_DEV) % N_DEV
        comm_ref[stage] = acc_ref[pl.ds(c0 * CHUNK, CHUNK), :]
        for s in range(N_DEV - 1):
            src = comm_ref.at[stage] if s == 0 else comm_ref.at[s - 1]
            rdma = pltpu.make_async_remote_copy(
                src_ref=src,
                dst_ref=comm_ref.at[s],
                send_sem=send_sems.at[s],
                recv_sem=recv_sems.at[s],
                device_id=(right,),
                device_id_type=pl.DeviceIdType.MESH,
            )
            rdma.start()
            rdma.wait()
            c = (my - s - 1 + N_DEV) % N_DEV
            comm_ref[s] = comm_ref[s] + acc_ref[pl.ds(c * CHUNK, CHUNK), :]

        own = (my + 1) % N_DEV
        out_ref[pl.ds(own * CHUNK, CHUNK), :] = comm_ref[N_DEV - 2]

        for t in range(N_DEV - 1):
            slot = (N_DEV - 1) + t
            src = comm_ref.at[slot - 1]
            rdma = pltpu.make_async_remote_copy(
                src_ref=src,
                dst_ref=comm_ref.at[slot],
                send_sem=send_sems.at[slot],
                recv_sem=recv_sems.at[slot],
                device_id=(right,),
                device_id_type=pl.DeviceIdType.MESH,
            )
            rdma.start()
            rdma.wait()
            c = (my - t + N_DEV) % N_DEV
            out_ref[pl.ds(c * CHUNK, CHUNK), :] = comm_ref[slot]

    return pl.pallas_call(
        body,
        out_shape=jax.ShapeDtypeStruct((m, n), jnp.float32),
        in_specs=[
            pl.BlockSpec(memory_space=pltpu.VMEM),
            pl.BlockSpec(memory_space=pltpu.VMEM),
            pl.BlockSpec(memory_space=pltpu.VMEM),
        ],
        out_specs=pl.BlockSpec(memory_space=pltpu.VMEM),
        scratch_shapes=[
            pltpu.VMEM((m, n), jnp.float32),
            pltpu.VMEM((N_HOPS + 1, CHUNK, n), jnp.float32),
            pltpu.SemaphoreType.DMA((N_HOPS,)),
            pltpu.SemaphoreType.DMA((N_HOPS,)),
        ],
        compiler_params=pltpu.CompilerParams(
            collective_id=0,
            vmem_limit_bytes=100 * 1024 * 1024,
        ),
    )(x, W1, W2)


# device time: 81029 ns/iter; 1.0633x vs baseline; 1.0633x over previous
import jax
import jax.numpy as jnp
from jax import lax
from jax.experimental import pallas as pl
from jax.experimental.pallas import tpu as pltpu

N_DEV = 8
M = 768
CHUNK = M // N_DEV
N_HOPS = 2 * (N_DEV - 1)


def kernel(x, W1, W2):
    m, k = x.shape
    _, h_per = W1.shape
    _, n = W2.shape

    def body(x_ref, w1_ref, w2_ref, out_ref, comm_ref, send_sems, recv_sems):
        my = lax.axis_index("i")
        r = jnp.where(my <= 3, my, 11 - my)
        succ = jnp.where(
            my == 3, 7,
            jnp.where(my == 4, 0, jnp.where(my <= 2, my + 1, my - 1)),
        )
        pred = jnp.where(
            my == 7, 3,
            jnp.where(my == 0, 4, jnp.where(my <= 3, my - 1, my + 1)),
        )

        barrier_sem = pltpu.get_barrier_semaphore()
        for nbr in (pred, succ):
            pl.semaphore_signal(
                barrier_sem, inc=1,
                device_id=(nbr,), device_id_type=pl.DeviceIdType.MESH,
            )
        pl.semaphore_wait(barrier_sem, 2)

        def compute_chunk(c):
            rows = pl.ds(c * CHUNK, CHUNK)
            hblk = jnp.maximum(
                jnp.dot(x_ref[rows, :], w1_ref[:, :],
                        preferred_element_type=jnp.float32),
                0.0,
            )
            return jnp.dot(hblk, w2_ref[:, :],
                           preferred_element_type=jnp.float32)

        def hop(s, src_slot):
            rdma = pltpu.make_async_remote_copy(
                src_ref=comm_ref.at[src_slot],
                dst_ref=comm_ref.at[s],
                send_sem=send_sems.at[s],
                recv_sem=recv_sems.at[s],
                device_id=(succ,),
                device_id_type=pl.DeviceIdType.MESH,
            )
            return rdma

        stage = N_HOPS
        comm_ref[stage] = compute_chunk(r)
        hop(0, stage).start()
        for s in range(N_DEV - 1):
            c = (r - s - 1 + N_DEV) % N_DEV
            part = compute_chunk(c)
            hop(s, stage if s == 0 else s - 1).wait_recv()
            comm_ref[s] = comm_ref[s] + part
            if s < N_DEV - 2:
                hop(s + 1, s).start()

        hop(N_DEV - 1, N_DEV - 2).start()
        own = (r + 1) % N_DEV
        out_ref[pl.ds(own * CHUNK, CHUNK), :] = comm_ref[N_DEV - 2]
        for t in range(N_DEV - 1):
            slot = (N_DEV - 1) + t
            hop(slot, slot - 1).wait_recv()
            if t < N_DEV - 2:
                hop(slot + 1, slot).start()
            c = (r - t + N_DEV) % N_DEV
            out_ref[pl.ds(c * CHUNK, CHUNK), :] = comm_ref[slot]

        for s in range(N_HOPS):
            hop(s, stage if s == 0 else s - 1).wait_send()

    return pl.pallas_call(
        body,
        out_shape=jax.ShapeDtypeStruct((m, n), jnp.float32),
        in_specs=[
            pl.BlockSpec(memory_space=pltpu.VMEM),
            pl.BlockSpec(memory_space=pltpu.VMEM),
            pl.BlockSpec(memory_space=pltpu.VMEM),
        ],
        out_specs=pl.BlockSpec(memory_space=pltpu.VMEM),
        scratch_shapes=[
            pltpu.VMEM((N_HOPS + 1, CHUNK, n), jnp.float32),
            pltpu.SemaphoreType.DMA((N_HOPS,)),
            pltpu.SemaphoreType.DMA((N_HOPS,)),
        ],
        compiler_params=pltpu.CompilerParams(
            collective_id=0,
            vmem_limit_bytes=100 * 1024 * 1024,
        ),
    )(x, W1, W2)


# device time: 42204 ns/iter; 2.0414x vs baseline; 1.9199x over previous
import os

import jax
import jax.numpy as jnp
from jax import lax
from jax.experimental import pallas as pl
from jax.experimental.pallas import tpu as pltpu

_VARIANT = os.environ.get("KERNEL_VARIANT", "full")

N_DEV = 8
M = 768
SLAB = M // 3
N_BF = 3
N_STAGE = 3
AX_X, AX_Y, AX_Z = 1, 3, 4
PERM = ((AX_Z, AX_Y, AX_X),
        (AX_Y, AX_X, AX_Z),
        (AX_X, AX_Z, AX_Y))


def kernel(x, W1, W2):
    m, k = x.shape
    _, h_per = W1.shape
    _, n = W2.shape

    def body(x_ref, w1_ref, w2_ref, out_ref, part_ref,
             rs0_ref, rs1_ref, rs2_ref, send_sems, recv_sems):
        my = lax.axis_index("i")

        def axis_bit(mask):
            if mask == AX_X:
                return jnp.bitwise_and(jnp.bitwise_xor(my, my >> 1), 1)
            if mask == AX_Y:
                return jnp.bitwise_and(my >> 1, 1)
            return jnp.bitwise_and(my >> 2, 1)

        def partner(mask):
            return jnp.bitwise_xor(my, mask)

        barrier_sem = pltpu.get_barrier_semaphore()
        for mask in (AX_X, AX_Y, AX_Z):
            pl.semaphore_signal(
                barrier_sem, inc=1,
                device_id=(partner(mask),),
                device_id_type=pl.DeviceIdType.MESH,
            )
        pl.semaphore_wait(barrier_sem, 3)

        def compute_rows(off, size):
            rows = pl.ds(off, size)
            hblk = jnp.maximum(
                jnp.dot(x_ref[rows, :], w1_ref[:, :],
                        preferred_element_type=jnp.float32),
                0.0,
            )
            return jnp.dot(hblk, w2_ref[:, :],
                           preferred_element_type=jnp.float32)

        if _VARIANT == "compute_only":
            for b in range(N_BF):
                out_ref[pl.ds(b * SLAB, SLAB), :] = compute_rows(
                    b * SLAB, SLAB)
            return

        rs_bufs = (rs0_ref, rs1_ref, rs2_ref)

        def rs_rdma(b, s, send_off, half):
            return pltpu.make_async_remote_copy(
                src_ref=part_ref.at[b, pl.ds(send_off, half), :],
                dst_ref=rs_bufs[s].at[b],
                send_sem=send_sems.at[b * N_STAGE + s],
                recv_sem=recv_sems.at[b * N_STAGE + s],
                device_id=(partner(PERM[b][s]),),
                device_id_type=pl.DeviceIdType.MESH,
            )

        def ag_rdma(b, s, cur_off, cur_sz):
            idx = 9 + b * N_STAGE + s
            return pltpu.make_async_remote_copy(
                src_ref=out_ref.at[pl.ds(cur_off, cur_sz), :],
                dst_ref=out_ref.at[pl.ds(cur_off, cur_sz), :],
                send_sem=send_sems.at[idx],
                recv_sem=recv_sems.at[idx],
                device_id=(partner(PERM[b][2 - s]),),
                device_id_type=pl.DeviceIdType.MESH,
            )

        offs = []
        sends = []
        for b in range(N_BF):
            part_ref[b, :, :] = compute_rows(b * SLAB, SLAB)
            bkeep = axis_bit(PERM[b][0])
            half = SLAB // 2
            send_off = (1 - bkeep) * half
            d = rs_rdma(b, 0, send_off, half)
            d.start()
            sends.append(d)
            offs.append(bkeep * half)

        ag_state = [None] * N_BF
        for s in range(N_STAGE):
            half = SLAB >> (s + 1)
            for b in range(N_BF):
                rs_rdma(b, s, 0, half).wait_recv()
                koff = offs[b]
                part_ref[b, pl.ds(koff, half), :] = (
                    part_ref[b, pl.ds(koff, half), :] + rs_bufs[s][b]
                )
                if s < N_STAGE - 1:
                    nhalf = half // 2
                    bkeep = axis_bit(PERM[b][s + 1])
                    send_off = koff + (1 - bkeep) * nhalf
                    d = rs_rdma(b, s + 1, send_off, nhalf)
                    d.start()
                    sends.append(d)
                    offs[b] = koff + bkeep * nhalf
                else:
                    cur_off = b * SLAB + koff
                    out_ref[pl.ds(cur_off, half), :] = part_ref[
                        b, pl.ds(koff, half), :]
                    d = ag_rdma(b, 0, cur_off, half)
                    d.start()
                    sends.append(d)
                    bkeep = axis_bit(PERM[b][2])
                    ag_state[b] = (cur_off - bkeep * half, half)

        for s in range(N_STAGE):
            for b in range(N_BF):
                par_off, sz = ag_state[b]
                ag_rdma(b, s, par_off, sz).wait_recv()
                if s < N_STAGE - 1:
                    d = ag_rdma(b, s + 1, par_off, 2 * sz)
                    d.start()
                    sends.append(d)
                    bkeep = axis_bit(PERM[b][2 - (s + 1)])
                    ag_state[b] = (par_off - bkeep * 2 * sz, 2 * sz)

        for d in sends:
            d.wait_send()

    return pl.pallas_call(
        body,
        out_shape=jax.ShapeDtypeStruct((m, n), jnp.float32),
        in_specs=[
            pl.BlockSpec(memory_space=pltpu.VMEM),
            pl.BlockSpec(memory_space=pltpu.VMEM),
            pl.BlockSpec(memory_space=pltpu.VMEM),
        ],
        out_specs=pl.BlockSpec(memory_space=pltpu.VMEM),
        scratch_shapes=[
            pltpu.VMEM((N_BF, SLAB, n), jnp.float32),
            pltpu.VMEM((N_BF, SLAB // 2, n), jnp.float32),
            pltpu.VMEM((N_BF, SLAB // 4, n), jnp.float32),
            pltpu.VMEM((N_BF, SLAB // 8, n), jnp.float32),
            pltpu.SemaphoreType.DMA((18,)),
            pltpu.SemaphoreType.DMA((18,)),
        ],
        compiler_params=pltpu.CompilerParams(
            collective_id=0,
            vmem_limit_bytes=100 * 1024 * 1024,
        ),
    )(x, W1, W2)
